# baseline (speedup 1.0000x reference)
import functools

import jax
import jax.numpy as jnp
from jax import lax
from jax.experimental import pallas as pl
from jax.experimental.pallas import tpu as pltpu
from jax.experimental.pallas import tpu_sc as plsc

B = 1024
S = 20
V = 100000
E = 32

NC = 2
NS = 16
NW = NC * NS
NTOK = B * S
TPW = NTOK // NW
CK = 128
NCHK = TPW // CK
WROWS = V * E // 128

_sc_mesh = plsc.VectorSubcoreMesh(core_axis_name="c", subcore_axis_name="s")


@functools.partial(
    pl.kernel,
    mesh=_sc_mesh,
    out_type=jax.ShapeDtypeStruct((NTOK * E,), jnp.float32),
    scratch_types=[
        pltpu.VMEM((TPW,), jnp.int32),
        pltpu.VMEM((TPW,), jnp.int32),
        pltpu.VMEM((CK, 128), jnp.float32),
        pltpu.VMEM((TPW * E,), jnp.float32),
        pltpu.SemaphoreType.DMA,
    ],
    compiler_params=pltpu.CompilerParams(use_tc_tiling_on_sc=True),
)
def _sc_gather(tok_hbm, table_hbm, out_hbm, tok_v, ridx_v, wide_v, out_v, sem):
    wid = lax.axis_index("s") * NC + lax.axis_index("c")
    pltpu.sync_copy(tok_hbm.at[pl.ds(wid * TPW, TPW)], tok_v)
    for g in range(TPW // 16):
        ridx_v[pl.ds(g * 16, 16)] = tok_v[pl.ds(g * 16, 16)] >> 2
    for k in range(NCHK):
        pltpu.async_copy(
            table_hbm.at[ridx_v.at[pl.ds(k * CK, CK)]], wide_v, sem
        ).wait()

        @pl.loop(0, CK // 16)
        def _extract(g):
            i0 = k * CK + g * 16
            off16 = (tok_v[pl.ds(i0, 16)] & 3) * E
            for l in range(16):
                off = off16[l]
                i = i0 + l
                for h in range(E // 16):
                    out_v[pl.ds(i * E + h * 16, 16)] = wide_v[
                        g * 16 + l, pl.ds(off + h * 16, 16)
                    ]

    pltpu.sync_copy(out_v, out_hbm.at[pl.ds(wid * TPW * E, TPW * E)])


TV = 2048
NV = (V + TV - 1) // TV


def _mlp_body(x_ref, w1_ref, b1_ref, w2_ref, b2_ref, outT_ref, hid_ref):
    @pl.when(pl.program_id(0) == 0)
    def _():
        hid_ref[...] = (
            jnp.dot(x_ref[...], w1_ref[...], preferred_element_type=jnp.float32)
            + b1_ref[...]
        )

    outT_ref[...] = (
        jax.lax.dot_general(
            w2_ref[...],
            hid_ref[...],
            (((0,), (1,)), ((), ())),
            preferred_element_type=jnp.float32,
        )
        + b2_ref[...].T
    )


def _tc_mlp(x, W1, b1, W2, b2):
    outT = pl.pallas_call(
        _mlp_body,
        grid=(NV,),
        in_specs=[
            pl.BlockSpec((B, S * E), lambda j: (0, 0)),
            pl.BlockSpec((S * E, E), lambda j: (0, 0)),
            pl.BlockSpec((1, E), lambda j: (0, 0)),
            pl.BlockSpec((E, TV), lambda j: (0, j)),
            pl.BlockSpec((1, TV), lambda j: (0, j)),
        ],
        out_specs=pl.BlockSpec((TV, B), lambda j: (j, 0)),
        out_shape=jax.ShapeDtypeStruct((V, B), jnp.float32),
        scratch_shapes=[pltpu.VMEM((B, E), jnp.float32)],
    )(x, W1, b1.reshape(1, E), W2, b2.reshape(1, V))
    return outT.T


def kernel(tokens, table, W1, b1, W2, b2):
    xflat = _sc_gather(tokens.reshape(NTOK), table.reshape(WROWS, 128))
    x = xflat.reshape(B, S * E)
    return _tc_mlp(x, W1, b1, W2, b2)


# speedup vs baseline: 1.0525x; 1.0525x over previous
import functools

import jax
import jax.numpy as jnp
from jax import lax
from jax.experimental import pallas as pl
from jax.experimental.pallas import tpu as pltpu
from jax.experimental.pallas import tpu_sc as plsc

B = 1024
S = 20
V = 100000
E = 32

NC = 2
NS = 16
NW = NC * NS
NTOK = B * S
TPW = NTOK // NW
CK = 128
NCHK = TPW // CK
_sc_mesh = plsc.VectorSubcoreMesh(core_axis_name="c", subcore_axis_name="s")


@functools.partial(
    pl.kernel,
    mesh=_sc_mesh,
    out_type=jax.ShapeDtypeStruct((NTOK, E), jnp.float32),
    scratch_types=[
        pltpu.VMEM((NCHK, CK), jnp.int32),
        pltpu.VMEM((TPW, E), jnp.float32),
        pltpu.SemaphoreType.DMA,
    ],
    compiler_params=pltpu.CompilerParams(use_tc_tiling_on_sc=False),
)
def _sc_gather(tok_hbm, table_hbm, out_hbm, idx_v, rows_v, sem):
    wid = lax.axis_index("s") * NC + lax.axis_index("c")
    pltpu.sync_copy(tok_hbm.at[wid], idx_v)
    copies = []
    for j in range(NCHK):
        copies.append(
            pltpu.async_copy(
                table_hbm.at[idx_v.at[j]],
                rows_v.at[pl.ds(j * CK, CK)],
                sem,
            )
        )
    for c in copies:
        c.wait()
    pltpu.sync_copy(rows_v, out_hbm.at[pl.ds(wid * TPW, TPW)])


TV = 4096
NV = (V + TV - 1) // TV


def _mlp_body(x_ref, w1_ref, b1_ref, w2_ref, b2_ref, outT_ref, hid_ref):
    @pl.when(pl.program_id(0) == 0)
    def _():
        hid_ref[...] = (
            jnp.dot(x_ref[...], w1_ref[...], preferred_element_type=jnp.float32)
            + b1_ref[...]
        )

    outT_ref[...] = (
        jax.lax.dot_general(
            w2_ref[...],
            hid_ref[...],
            (((0,), (1,)), ((), ())),
            preferred_element_type=jnp.float32,
        )
        + b2_ref[...].T
    )


def _tc_mlp(x, W1, b1, W2, b2):
    outT = pl.pallas_call(
        _mlp_body,
        grid=(NV,),
        in_specs=[
            pl.BlockSpec((B, S * E), lambda j: (0, 0)),
            pl.BlockSpec((S * E, E), lambda j: (0, 0)),
            pl.BlockSpec((1, E), lambda j: (0, 0)),
            pl.BlockSpec((E, TV), lambda j: (0, j)),
            pl.BlockSpec((1, TV), lambda j: (0, j)),
        ],
        out_specs=pl.BlockSpec((TV, B), lambda j: (j, 0)),
        out_shape=jax.ShapeDtypeStruct((V, B), jnp.float32),
        scratch_shapes=[pltpu.VMEM((B, E), jnp.float32)],
    )(x, W1, b1.reshape(1, E), W2, b2.reshape(1, V))
    return outT.T


def kernel(tokens, table, W1, b1, W2, b2):
    x = _sc_gather(tokens.reshape(NW, NCHK, CK), table)
    x = x.reshape(B, S * E)
    return _tc_mlp(x, W1, b1, W2, b2)
